# baseline (device time: 296439 ns/iter reference)
import jax
import jax.numpy as jnp
from jax import lax
from jax.experimental import pallas as pl
from jax.experimental.pallas import tpu as pltpu

N_DEV = 4
M_PER = 2048
D = 2048
DH = D // 2
NSUB = 4
RS = M_PER // NSUB
EPS = 1e-6


def kernel(partial, gamma):
    gamma2d = gamma.reshape(1, D)

    def body(p_ref, g_ref, o_ref,
             cw_slots, ccw_slots, tmp_cw, tmp_ccw,
             send_sems, recv_sems, dsems, seed_sems,
             credit_cw, credit_ccw):
        my = lax.axis_index("i")
        left = lax.rem(my + N_DEV - 1, N_DEV)
        right = lax.rem(my + 1, N_DEV)

        def rows(q):
            return pl.ds(q * RS, RS)

        def chunk_lo(c, q0, nq):
            return p_ref.at[0, pl.ds(c * M_PER + q0 * RS, nq * RS),
                            pl.ds(0, DH)]

        def chunk_hi(c, q0, nq):
            return p_ref.at[0, pl.ds(c * M_PER + q0 * RS, nq * RS),
                            pl.ds(DH, DH)]

        def rdma(dir_idx, h, q):
            slots = cw_slots if dir_idx == 0 else ccw_slots
            return pltpu.make_async_remote_copy(
                src_ref=slots.at[h % 2, rows(q), :],
                dst_ref=slots.at[(h + 1) % 2, rows(q), :],
                send_sem=send_sems.at[dir_idx, q],
                recv_sem=recv_sems.at[dir_idx, (h + 1) % 2, q],
                device_id=(right if dir_idx == 0 else left,),
                device_id_type=pl.DeviceIdType.MESH,
            )

        def prefetch(c_cw, c_ccw, q0):
            a = pltpu.make_async_copy(chunk_lo(c_cw, q0, 2), tmp_cw,
                                      dsems.at[0])
            b = pltpu.make_async_copy(chunk_hi(c_ccw, q0, 2), tmp_ccw,
                                      dsems.at[1])
            a.start()
            b.start()
            return a, b

        seeds = []
        for q in range(NSUB):
            sc = pltpu.make_async_copy(chunk_lo(left, q, 1),
                                       cw_slots.at[0, rows(q), :],
                                       seed_sems.at[0, q])
            sd = pltpu.make_async_copy(chunk_hi(right, q, 1),
                                       ccw_slots.at[0, rows(q), :],
                                       seed_sems.at[1, q])
            sc.start()
            sd.start()
            seeds.append((sc, sd))

        barrier = pltpu.get_barrier_semaphore()
        for nbr in (left, right):
            pl.semaphore_signal(
                barrier, inc=1,
                device_id=(nbr,), device_id_type=pl.DeviceIdType.MESH,
            )
        pl.semaphore_wait(barrier, 2)

        for q in range(NSUB):
            seeds[q][0].wait()
            seeds[q][1].wait()
            rdma(0, 0, q).start()
            rdma(1, 0, q).start()

        c_cw = lax.rem(my + 2 * N_DEV - 2, N_DEV)
        c_ccw = lax.rem(my + 2, N_DEV)
        p0, p1 = prefetch(c_cw, c_ccw, 0)

        for h in range(N_DEV - 1):
            r = (h + 1) % 2
            for q in range(NSUB):
                d_cw = rdma(0, h, q)
                d_ccw = rdma(1, h, q)
                d_cw.wait_send()
                d_ccw.wait_send()
                if h < N_DEV - 2:
                    pl.semaphore_signal(
                        credit_cw.at[q], inc=1,
                        device_id=(left,),
                        device_id_type=pl.DeviceIdType.MESH,
                    )
                    pl.semaphore_signal(
                        credit_ccw.at[q], inc=1,
                        device_id=(right,),
                        device_id_type=pl.DeviceIdType.MESH,
                    )
                d_cw.wait_recv()
                d_ccw.wait_recv()
                if q % 2 == 0:
                    p0.wait()
                    p1.wait()
                t = pl.ds((q % 2) * RS, RS)
                cw_slots[r, rows(q), :] = (cw_slots[r, rows(q), :]
                                           + tmp_cw[t, :])
                ccw_slots[r, rows(q), :] = (ccw_slots[r, rows(q), :]
                                            + tmp_ccw[t, :])
                if q == 1:
                    p0, p1 = prefetch(c_cw, c_ccw, 2)
                elif q == 3 and h < N_DEV - 2:
                    c_cw = lax.rem(my + 2 * N_DEV - 3 - h, N_DEV)
                    c_ccw = lax.rem(my + 3 + h, N_DEV)
                    p0, p1 = prefetch(c_cw, c_ccw, 0)
                if h < N_DEV - 2:
                    pl.semaphore_wait(credit_cw.at[q], 1)
                    pl.semaphore_wait(credit_ccw.at[q], 1)
                    rdma(0, h + 1, q).start()
                    rdma(1, h + 1, q).start()
                else:
                    a = cw_slots[1, rows(q), :]
                    b = ccw_slots[1, rows(q), :]
                    ssq = (jnp.sum(a * a, axis=-1, keepdims=True)
                           + jnp.sum(b * b, axis=-1, keepdims=True))
                    inv = lax.rsqrt(ssq / D + EPS)
                    o_ref[rows(q), pl.ds(0, DH)] = a * inv * g_ref[:, 0:DH]
                    o_ref[rows(q), pl.ds(DH, DH)] = b * inv * g_ref[:, DH:D]

    return pl.pallas_call(
        body,
        out_shape=jax.ShapeDtypeStruct((M_PER, D), jnp.float32),
        in_specs=[
            pl.BlockSpec(memory_space=pl.ANY),
            pl.BlockSpec(memory_space=pltpu.VMEM),
        ],
        out_specs=pl.BlockSpec(memory_space=pltpu.VMEM),
        scratch_shapes=[
            pltpu.VMEM((2, M_PER, DH), jnp.float32),
            pltpu.VMEM((2, M_PER, DH), jnp.float32),
            pltpu.VMEM((2 * RS, DH), jnp.float32),
            pltpu.VMEM((2 * RS, DH), jnp.float32),
            pltpu.SemaphoreType.DMA((2, NSUB)),
            pltpu.SemaphoreType.DMA((2, 2, NSUB)),
            pltpu.SemaphoreType.DMA((2,)),
            pltpu.SemaphoreType.DMA((2, NSUB)),
            pltpu.SemaphoreType.REGULAR((NSUB,)),
            pltpu.SemaphoreType.REGULAR((NSUB,)),
        ],
        compiler_params=pltpu.CompilerParams(
            collective_id=0, vmem_limit_bytes=63 * 1024 * 1024),
    )(partial, gamma2d)


# device time: 291880 ns/iter; 1.0156x vs baseline; 1.0156x over previous
import jax
import jax.numpy as jnp
from jax import lax
from jax.experimental import pallas as pl
from jax.experimental.pallas import tpu as pltpu

N_DEV = 4
M_PER = 2048
D = 2048
DH = D // 2
NSUB = 4
RS = M_PER // NSUB
EPS = 1e-6


def kernel(partial, gamma):
    gamma2d = gamma.reshape(1, D)

    def body(p_ref, g_ref, o_ref,
             cw_slots, ccw_slots, tmp_cw, tmp_ccw,
             send_sems, recv_sems, dsems, out_sems,
             credit_cw, credit_ccw):
        my = lax.axis_index("i")
        left = lax.rem(my + N_DEV - 1, N_DEV)
        right = lax.rem(my + 1, N_DEV)

        def rows(q):
            return pl.ds(q * RS, RS)

        def chunk_lo(c, q):
            return p_ref.at[0, pl.ds(c * M_PER + q * RS, RS), pl.ds(0, DH)]

        def chunk_hi(c, q):
            return p_ref.at[0, pl.ds(c * M_PER + q * RS, RS), pl.ds(DH, DH)]

        def rdma(dir_idx, h, q):
            slots = cw_slots if dir_idx == 0 else ccw_slots
            if h == 0:
                src = (chunk_lo(left, q) if dir_idx == 0
                       else chunk_hi(right, q))
            else:
                src = slots.at[h % 2, rows(q), :]
            return pltpu.make_async_remote_copy(
                src_ref=src,
                dst_ref=slots.at[(h + 1) % 2, rows(q), :],
                send_sem=send_sems.at[dir_idx, q],
                recv_sem=recv_sems.at[dir_idx, (h + 1) % 2, q],
                device_id=(right if dir_idx == 0 else left,),
                device_id_type=pl.DeviceIdType.MESH,
            )

        barrier = pltpu.get_barrier_semaphore()
        for nbr in (left, right):
            pl.semaphore_signal(
                barrier, inc=1,
                device_id=(nbr,), device_id_type=pl.DeviceIdType.MESH,
            )
        pl.semaphore_wait(barrier, 2)

        for q in range(NSUB):
            rdma(0, 0, q).start()
            rdma(1, 0, q).start()

        c_cw = lax.rem(my + 2 * N_DEV - 2, N_DEV)
        c_ccw = lax.rem(my + 2, N_DEV)
        p0 = pltpu.make_async_copy(
            p_ref.at[0, pl.ds(c_cw * M_PER, M_PER), pl.ds(0, DH)],
            tmp_cw, dsems.at[0])
        p1 = pltpu.make_async_copy(
            p_ref.at[0, pl.ds(c_ccw * M_PER, M_PER), pl.ds(DH, DH)],
            tmp_ccw, dsems.at[1])
        p0.start()
        p1.start()

        out_copies = []
        for h in range(N_DEV - 1):
            r = (h + 1) % 2
            for q in range(NSUB):
                d_cw = rdma(0, h, q)
                d_ccw = rdma(1, h, q)
                d_cw.wait_send()
                d_ccw.wait_send()
                if h == 1:
                    pl.semaphore_signal(
                        credit_cw.at[q], inc=1,
                        device_id=(left,),
                        device_id_type=pl.DeviceIdType.MESH,
                    )
                    pl.semaphore_signal(
                        credit_ccw.at[q], inc=1,
                        device_id=(right,),
                        device_id_type=pl.DeviceIdType.MESH,
                    )
                d_cw.wait_recv()
                d_ccw.wait_recv()
                if q == 0:
                    p0.wait()
                    p1.wait()
                cw_slots[r, rows(q), :] = (cw_slots[r, rows(q), :]
                                           + tmp_cw[rows(q), :])
                ccw_slots[r, rows(q), :] = (ccw_slots[r, rows(q), :]
                                            + tmp_ccw[rows(q), :])
                if h < N_DEV - 2:
                    if h == 1:
                        pl.semaphore_wait(credit_cw.at[q], 1)
                        pl.semaphore_wait(credit_ccw.at[q], 1)
                    rdma(0, h + 1, q).start()
                    rdma(1, h + 1, q).start()
                else:
                    a = cw_slots[1, rows(q), :]
                    b = ccw_slots[1, rows(q), :]
                    ssq = (jnp.sum(a * a, axis=-1, keepdims=True)
                           + jnp.sum(b * b, axis=-1, keepdims=True))
                    inv = lax.rsqrt(ssq / D + EPS)
                    cw_slots[0, rows(q), :] = a * inv * g_ref[:, 0:DH]
                    ccw_slots[0, rows(q), :] = b * inv * g_ref[:, DH:D]
                    oc = pltpu.make_async_copy(
                        cw_slots.at[0, rows(q), :],
                        o_ref.at[rows(q), pl.ds(0, DH)],
                        out_sems.at[0, q])
                    od = pltpu.make_async_copy(
                        ccw_slots.at[0, rows(q), :],
                        o_ref.at[rows(q), pl.ds(DH, DH)],
                        out_sems.at[1, q])
                    oc.start()
                    od.start()
                    out_copies += [oc, od]
            if h < N_DEV - 2:
                c_cw = lax.rem(my + 2 * N_DEV - 3 - h, N_DEV)
                c_ccw = lax.rem(my + 3 + h, N_DEV)
                p0 = pltpu.make_async_copy(
                    p_ref.at[0, pl.ds(c_cw * M_PER, M_PER), pl.ds(0, DH)],
                    tmp_cw, dsems.at[0])
                p1 = pltpu.make_async_copy(
                    p_ref.at[0, pl.ds(c_ccw * M_PER, M_PER), pl.ds(DH, DH)],
                    tmp_ccw, dsems.at[1])
                p0.start()
                p1.start()

        for c in out_copies:
            c.wait()

    return pl.pallas_call(
        body,
        out_shape=jax.ShapeDtypeStruct((M_PER, D), jnp.float32),
        in_specs=[
            pl.BlockSpec(memory_space=pl.ANY),
            pl.BlockSpec(memory_space=pltpu.VMEM),
        ],
        out_specs=pl.BlockSpec(memory_space=pltpu.MemorySpace.HBM),
        scratch_shapes=[
            pltpu.VMEM((2, M_PER, DH), jnp.float32),
            pltpu.VMEM((2, M_PER, DH), jnp.float32),
            pltpu.VMEM((M_PER, DH), jnp.float32),
            pltpu.VMEM((M_PER, DH), jnp.float32),
            pltpu.SemaphoreType.DMA((2, NSUB)),
            pltpu.SemaphoreType.DMA((2, 2, NSUB)),
            pltpu.SemaphoreType.DMA((2,)),
            pltpu.SemaphoreType.DMA((2, NSUB)),
            pltpu.SemaphoreType.REGULAR((NSUB,)),
            pltpu.SemaphoreType.REGULAR((NSUB,)),
        ],
        compiler_params=pltpu.CompilerParams(
            collective_id=0, vmem_limit_bytes=63 * 1024 * 1024),
    )(partial, gamma2d)


# device time: 291227 ns/iter; 1.0179x vs baseline; 1.0022x over previous
import jax
import jax.numpy as jnp
from jax import lax
from jax.experimental import pallas as pl
from jax.experimental.pallas import tpu as pltpu

N_DEV = 4
M_PER = 2048
D = 2048
DH = D // 2
NSUB = 8
RS = M_PER // NSUB
EPS = 1e-6


def kernel(partial, gamma):
    gamma2d = gamma.reshape(1, D)

    def body(p_ref, g_ref, o_ref,
             cw_slots, ccw_slots, tmp_cw, tmp_ccw,
             send_sems, recv_sems, dsems, out_sems,
             credit_cw, credit_ccw):
        my = lax.axis_index("i")
        left = lax.rem(my + N_DEV - 1, N_DEV)
        right = lax.rem(my + 1, N_DEV)

        def rows(q):
            return pl.ds(q * RS, RS)

        def chunk_lo(c, q):
            return p_ref.at[0, pl.ds(c * M_PER + q * RS, RS), pl.ds(0, DH)]

        def chunk_hi(c, q):
            return p_ref.at[0, pl.ds(c * M_PER + q * RS, RS), pl.ds(DH, DH)]

        def rdma(dir_idx, h, q):
            slots = cw_slots if dir_idx == 0 else ccw_slots
            if h == 0:
                src = (chunk_lo(left, q) if dir_idx == 0
                       else chunk_hi(right, q))
            else:
                src = slots.at[h % 2, rows(q), :]
            return pltpu.make_async_remote_copy(
                src_ref=src,
                dst_ref=slots.at[(h + 1) % 2, rows(q), :],
                send_sem=send_sems.at[dir_idx, q],
                recv_sem=recv_sems.at[dir_idx, (h + 1) % 2, q],
                device_id=(right if dir_idx == 0 else left,),
                device_id_type=pl.DeviceIdType.MESH,
            )

        barrier = pltpu.get_barrier_semaphore()
        for nbr in (left, right):
            pl.semaphore_signal(
                barrier, inc=1,
                device_id=(nbr,), device_id_type=pl.DeviceIdType.MESH,
            )
        pl.semaphore_wait(barrier, 2)

        for q in range(NSUB):
            rdma(0, 0, q).start()
            rdma(1, 0, q).start()

        c_cw = lax.rem(my + 2 * N_DEV - 2, N_DEV)
        c_ccw = lax.rem(my + 2, N_DEV)
        p0 = pltpu.make_async_copy(
            p_ref.at[0, pl.ds(c_cw * M_PER, M_PER), pl.ds(0, DH)],
            tmp_cw, dsems.at[0])
        p1 = pltpu.make_async_copy(
            p_ref.at[0, pl.ds(c_ccw * M_PER, M_PER), pl.ds(DH, DH)],
            tmp_ccw, dsems.at[1])
        p0.start()
        p1.start()

        out_copies = []
        for h in range(N_DEV - 1):
            r = (h + 1) % 2
            for q in range(NSUB):
                d_cw = rdma(0, h, q)
                d_ccw = rdma(1, h, q)
                d_cw.wait_send()
                d_ccw.wait_send()
                if h == 1:
                    pl.semaphore_signal(
                        credit_cw.at[q], inc=1,
                        device_id=(left,),
                        device_id_type=pl.DeviceIdType.MESH,
                    )
                    pl.semaphore_signal(
                        credit_ccw.at[q], inc=1,
                        device_id=(right,),
                        device_id_type=pl.DeviceIdType.MESH,
                    )
                d_cw.wait_recv()
                d_ccw.wait_recv()
                if q == 0:
                    p0.wait()
                    p1.wait()
                cw_slots[r, rows(q), :] = (cw_slots[r, rows(q), :]
                                           + tmp_cw[rows(q), :])
                ccw_slots[r, rows(q), :] = (ccw_slots[r, rows(q), :]
                                            + tmp_ccw[rows(q), :])
                if h < N_DEV - 2:
                    if h == 1:
                        pl.semaphore_wait(credit_cw.at[q], 1)
                        pl.semaphore_wait(credit_ccw.at[q], 1)
                    rdma(0, h + 1, q).start()
                    rdma(1, h + 1, q).start()
                else:
                    a = cw_slots[1, rows(q), :]
                    b = ccw_slots[1, rows(q), :]
                    ssq = (jnp.sum(a * a, axis=-1, keepdims=True)
                           + jnp.sum(b * b, axis=-1, keepdims=True))
                    inv = lax.rsqrt(ssq / D + EPS)
                    cw_slots[0, rows(q), :] = a * inv * g_ref[:, 0:DH]
                    ccw_slots[0, rows(q), :] = b * inv * g_ref[:, DH:D]
                    oc = pltpu.make_async_copy(
                        cw_slots.at[0, rows(q), :],
                        o_ref.at[rows(q), pl.ds(0, DH)],
                        out_sems.at[0, q])
                    od = pltpu.make_async_copy(
                        ccw_slots.at[0, rows(q), :],
                        o_ref.at[rows(q), pl.ds(DH, DH)],
                        out_sems.at[1, q])
                    oc.start()
                    od.start()
                    out_copies += [oc, od]
            if h < N_DEV - 2:
                c_cw = lax.rem(my + 2 * N_DEV - 3 - h, N_DEV)
                c_ccw = lax.rem(my + 3 + h, N_DEV)
                p0 = pltpu.make_async_copy(
                    p_ref.at[0, pl.ds(c_cw * M_PER, M_PER), pl.ds(0, DH)],
                    tmp_cw, dsems.at[0])
                p1 = pltpu.make_async_copy(
                    p_ref.at[0, pl.ds(c_ccw * M_PER, M_PER), pl.ds(DH, DH)],
                    tmp_ccw, dsems.at[1])
                p0.start()
                p1.start()

        for c in out_copies:
            c.wait()

    return pl.pallas_call(
        body,
        out_shape=jax.ShapeDtypeStruct((M_PER, D), jnp.float32),
        in_specs=[
            pl.BlockSpec(memory_space=pl.ANY),
            pl.BlockSpec(memory_space=pltpu.VMEM),
        ],
        out_specs=pl.BlockSpec(memory_space=pltpu.MemorySpace.HBM),
        scratch_shapes=[
            pltpu.VMEM((2, M_PER, DH), jnp.float32),
            pltpu.VMEM((2, M_PER, DH), jnp.float32),
            pltpu.VMEM((M_PER, DH), jnp.float32),
            pltpu.VMEM((M_PER, DH), jnp.float32),
            pltpu.SemaphoreType.DMA((2, NSUB)),
            pltpu.SemaphoreType.DMA((2, 2, NSUB)),
            pltpu.SemaphoreType.DMA((2,)),
            pltpu.SemaphoreType.DMA((2, NSUB)),
            pltpu.SemaphoreType.REGULAR((NSUB,)),
            pltpu.SemaphoreType.REGULAR((NSUB,)),
        ],
        compiler_params=pltpu.CompilerParams(
            collective_id=0, vmem_limit_bytes=63 * 1024 * 1024),
    )(partial, gamma2d)
